# baseline (speedup 1.0000x reference)
import dataclasses
import functools

import jax
import jax.numpy as jnp
from jax import lax
from jax.experimental import pallas as pl
from jax.experimental.pallas import tpu as pltpu
from jax.experimental.pallas import tpu_sc as plsc

N = 10000
D = 128
NC = 2
NS = 16
LANES = 16
NW = NC * NS
CHUNK = 128
NPAD = 10240
RPT = NPAD // NS
NROWBUF = 2
NEBUF = 8
AHEAD = 1
IDIST = 6
NEG_SLOPE = 0.01
EPS = 1e-5

_SC_PARAMS = pltpu.CompilerParams()
if "needs_layout_passes" in pltpu.CompilerParams.__dataclass_fields__:
    _SC_PARAMS = dataclasses.replace(_SC_PARAMS, needs_layout_passes=False)


def _propagate_sc(h, edata, zeros, epad):
    ept = epad // NW
    nchunk = ept // CHUNK
    assert nchunk % NEBUF == 0 and nchunk // NEBUF >= 2
    nouter = nchunk // NEBUF
    mesh = plsc.VectorSubcoreMesh(core_axis_name="c", subcore_axis_name="s")

    @functools.partial(
        pl.kernel,
        out_type=jax.ShapeDtypeStruct((NC, NPAD, D), jnp.float32),
        mesh=mesh,
        compiler_params=_SC_PARAMS,
        scratch_types=(
            [pltpu.VMEM_SHARED((NPAD, D), jnp.float32)]
            + [pltpu.VMEM((CHUNK, D), jnp.float32)] * NROWBUF
            + [pltpu.VMEM((3, CHUNK), jnp.int32)] * NEBUF
            + [pltpu.SemaphoreType.DMA] * (NEBUF + 2 * NROWBUF)
        ),
    )
    def k(h_hbm, e_hbm, z_hbm, out_hbm, acc, *sc):
        rows = sc[:NROWBUF]
        ebuf = sc[NROWBUF:NROWBUF + NEBUF]
        sems = sc[NROWBUF + NEBUF:]
        sem_e = sems[:NEBUF]
        sem_g = sems[NEBUF:NEBUF + NROWBUF]
        sem_s = sems[NEBUF + NROWBUF:]

        c = lax.axis_index("c")
        s = lax.axis_index("s")
        wid = c * NS + s
        chunk0 = wid * nchunk

        pltpu.sync_copy(z_hbm.at[pl.ds(s * RPT, RPT)],
                        acc.at[pl.ds(s * RPT, RPT)])
        plsc.subcore_barrier()

        def e_desc(chunk, eb):
            return pltpu.make_async_copy(
                e_hbm.at[chunk0 + chunk], ebuf[eb], sem_e[eb])

        def g_desc(rb, eb):
            return pltpu.make_async_copy(
                h_hbm.at[ebuf[eb].at[0]], rows[rb], sem_g[rb])

        def s_desc(rb, eb):
            return pltpu.make_async_copy(
                rows[rb], acc.at[ebuf[eb].at[1]], sem_s[rb])

        def mul_scat(rb, eb):
            rv, ev = rows[rb], ebuf[eb]

            @plsc.parallel_loop(0, CHUNK, unroll=2)
            def _(e):
                sel = jnp.zeros((LANES,), jnp.int32) + e
                two = jnp.full((LANES,), 2, jnp.int32)
                wb = plsc.bitcast(plsc.load_gather(ev, [two, sel]),
                                  jnp.float32)
                for q in range(D // LANES):
                    seg = rv[e, pl.ds(q * LANES, LANES)]
                    rv[e, pl.ds(q * LANES, LANES)] = seg * wb

            pltpu.async_copy(rv, acc.at[ev.at[1]], sem_s[rb], add=True)

        lag = NROWBUF - AHEAD
        def slot(cnum, jj, wait_s, do_idx, do_a):
            rb = jj % NROWBUF
            rba = (jj + AHEAD) % NROWBUF
            eba = (jj + AHEAD) % NEBUF
            ebl = (jj - lag) % NEBUF
            ebi = (jj + IDIST) % NEBUF
            if wait_s:
                s_desc(rba, ebl).wait()
            if do_a:
                if do_idx:
                    e_desc(cnum + IDIST, ebi).start()
                e_desc(0, eba).wait()
                g_desc(rba, eba).start()
            g_desc(rb, jj % NEBUF).wait()
            mul_scat(rb, jj % NEBUF)

        for q in range(IDIST):
            e_desc(q, q).start()
        for q in range(AHEAD):
            e_desc(0, q).wait()
            g_desc(q, q).start()
        for j in range(NEBUF):
            slot(j, j, wait_s=(j >= lag), do_idx=True, do_a=True)

        @pl.loop(1, nouter - 1)
        def _(t):
            cb = t * NEBUF
            for j in range(NEBUF):
                slot(cb + j, j, wait_s=True, do_idx=True, do_a=True)

        cb = (nouter - 1) * NEBUF
        for j in range(NEBUF):
            cnum = cb + j
            slot(cnum, j, wait_s=True,
                 do_idx=(cnum + IDIST < nchunk), do_a=(cnum + AHEAD < nchunk))
        for q in range(nchunk - lag, nchunk):
            s_desc(q % NROWBUF, q % NEBUF).wait()

        plsc.subcore_barrier()
        pltpu.sync_copy(acc.at[pl.ds(s * RPT, RPT)],
                        out_hbm.at[c, pl.ds(s * RPT, RPT)])

    return k(h, edata, zeros)


def _dense_tc(agg, h, W, b, bn):
    out_dim = W.shape[1]

    def body(agg_ref, h_ref, w_ref, b_ref, o_ref):
        a = agg_ref[0, :N, :] + agg_ref[1, :N, :] + h_ref[...]
        y = jnp.dot(a, w_ref[...], preferred_element_type=jnp.float32)
        y = y + b_ref[...]
        if bn:
            m = jnp.mean(y, axis=0, keepdims=True)
            v = jnp.mean((y - m) ** 2, axis=0, keepdims=True)
            y = (y - m) * lax.rsqrt(v + EPS)
            y = jnp.where(y >= 0.0, y, NEG_SLOPE * y)
        o_ref[...] = y

    return pl.pallas_call(
        body,
        out_shape=jax.ShapeDtypeStruct((N, out_dim), jnp.float32),
    )(agg, h, W, b.reshape(1, out_dim))


def kernel(x, edge_index, edge_weight, batch, W1, b1, W2, b2, W3, b3):
    e = edge_index.shape[1]
    grain = NW * CHUNK * NEBUF
    epad = ((e + grain - 1) // grain) * grain
    pad = epad - e
    src = jnp.concatenate([edge_index[0], jnp.zeros((pad,), jnp.int32)])
    dst = jnp.concatenate([edge_index[1], jnp.zeros((pad,), jnp.int32)])
    w = jnp.concatenate([edge_weight, jnp.zeros((pad,), jnp.float32)])
    nct = epad // CHUNK
    edata = jnp.stack(
        [src.reshape(nct, CHUNK),
         dst.reshape(nct, CHUNK),
         lax.bitcast_convert_type(w, jnp.int32).reshape(nct, CHUNK)],
        axis=1)
    zeros = jnp.zeros((NPAD, D), jnp.float32)

    h = x
    agg = _propagate_sc(h, edata, zeros, epad)
    h = _dense_tc(agg, h, W1, b1, True)
    agg = _propagate_sc(h, edata, zeros, epad)
    h = _dense_tc(agg, h, W2, b2, True)
    agg = _propagate_sc(h, edata, zeros, epad)
    return _dense_tc(agg, h, W3, b3, False)


# speedup vs baseline: 1.0727x; 1.0727x over previous
import dataclasses
import functools

import jax
import jax.numpy as jnp
from jax import lax
from jax.experimental import pallas as pl
from jax.experimental.pallas import tpu as pltpu
from jax.experimental.pallas import tpu_sc as plsc

N = 10000
D = 128
NC = 2
NS = 16
LANES = 16
CHUNK = 128
NPAD = 10240
RPT = NPAD // NS
NROWBUF = 2
NEBUF = 8
AHEAD = 1
IDIST = 6
NEG_SLOPE = 0.01
EPS = 1e-5

_SC_PARAMS = pltpu.CompilerParams()
if "needs_layout_passes" in pltpu.CompilerParams.__dataclass_fields__:
    _SC_PARAMS = dataclasses.replace(_SC_PARAMS, needs_layout_passes=False)


def _propagate_sc(h, edata, zeros, epad):
    ept = epad // (NC * NS)
    nchunk = ept // CHUNK
    assert nchunk % NEBUF == 0 and nchunk // NEBUF >= 2
    nouter = nchunk // NEBUF
    mesh = plsc.VectorSubcoreMesh(core_axis_name="c", subcore_axis_name="s")

    @functools.partial(
        pl.kernel,
        out_type=jax.ShapeDtypeStruct((NC, NPAD, D), jnp.float32),
        mesh=mesh,
        compiler_params=_SC_PARAMS,
        scratch_types=(
            [pltpu.VMEM_SHARED((NPAD, D), jnp.float32)]
            + [pltpu.VMEM((CHUNK, D), jnp.float32)] * NROWBUF
            + [pltpu.VMEM((3, CHUNK), jnp.int32)] * NEBUF
            + [pltpu.SemaphoreType.DMA] * (NEBUF + NROWBUF)
        ),
    )
    def k(h_hbm, e_hbm, z_hbm, out_hbm, acc, *sc):
        rows = sc[:NROWBUF]
        ebuf = sc[NROWBUF:NROWBUF + NEBUF]
        sems = sc[NROWBUF + NEBUF:]
        sem_e = sems[:NEBUF]
        sem_g = sems[NEBUF:]

        c = lax.axis_index("c")
        s = lax.axis_index("s")
        chunk0 = (c * NS + s) * nchunk

        slab = pl.ds(s * RPT, RPT)
        pltpu.sync_copy(z_hbm.at[slab], acc.at[slab])
        plsc.subcore_barrier()

        def e_desc(chunk, eb):
            return pltpu.make_async_copy(
                e_hbm.at[chunk0 + chunk], ebuf[eb], sem_e[eb])

        def g_desc(rb, eb):
            return pltpu.make_async_copy(
                h_hbm.at[ebuf[eb].at[0]], rows[rb], sem_g[rb])

        def mul_scat(rb, eb):
            rv, ev = rows[rb], ebuf[eb]

            @plsc.parallel_loop(0, CHUNK, unroll=2)
            def _(e):
                sel = jnp.zeros((LANES,), jnp.int32) + e
                two = jnp.full((LANES,), 2, jnp.int32)
                wb = plsc.bitcast(plsc.load_gather(ev, [two, sel]),
                                  jnp.float32)
                for q in range(D // LANES):
                    seg = rv[e, pl.ds(q * LANES, LANES)]
                    rv[e, pl.ds(q * LANES, LANES)] = seg * wb

            pltpu.sync_copy(rv, acc.at[ev.at[1]], add=True)

        def slot(cnum, jj, do_idx, do_a):
            rb = jj % NROWBUF
            rba = (jj + AHEAD) % NROWBUF
            eba = (jj + AHEAD) % NEBUF
            ebi = (jj + IDIST) % NEBUF
            if do_a:
                if do_idx:
                    e_desc(cnum + IDIST, ebi).start()
                e_desc(0, eba).wait()
                g_desc(rba, eba).start()
            g_desc(rb, jj % NEBUF).wait()
            mul_scat(rb, jj % NEBUF)

        for q in range(IDIST):
            e_desc(q, q).start()
        for q in range(AHEAD):
            e_desc(0, q).wait()
            g_desc(q, q).start()
        for j in range(NEBUF):
            slot(j, j, do_idx=True, do_a=True)

        @pl.loop(1, nouter - 1)
        def _(t):
            cb = t * NEBUF
            for j in range(NEBUF):
                slot(cb + j, j, do_idx=True, do_a=True)

        cb = (nouter - 1) * NEBUF
        for j in range(NEBUF):
            cnum = cb + j
            slot(cnum, j,
                 do_idx=(cnum + IDIST < nchunk), do_a=(cnum + AHEAD < nchunk))

        plsc.subcore_barrier()
        pltpu.sync_copy(acc.at[slab], out_hbm.at[c, slab])

    return k(h, edata, zeros)


def _dense_tc(agg, h, W, b, bn, pad_out):
    out_dim = W.shape[1]

    def body(agg_ref, h_ref, w_ref, b_ref, o_ref):
        a = agg_ref[0, :N, :] + agg_ref[1, :N, :] + h_ref[:N, :]
        y = jnp.dot(a, w_ref[...], preferred_element_type=jnp.float32)
        y = y + b_ref[...]
        if bn:
            m = jnp.mean(y, axis=0, keepdims=True)
            v = jnp.mean((y - m) ** 2, axis=0, keepdims=True)
            y = (y - m) * lax.rsqrt(v + EPS)
            y = jnp.where(y >= 0.0, y, NEG_SLOPE * y)
        if pad_out:
            o_ref[:N, :] = y
            o_ref[N:, :] = jnp.zeros((NPAD - N, out_dim), jnp.float32)
        else:
            o_ref[...] = y

    out_shape = ((NPAD, out_dim) if pad_out else (N, out_dim))
    return pl.pallas_call(
        body,
        out_shape=jax.ShapeDtypeStruct(out_shape, jnp.float32),
    )(agg, h, W, b.reshape(1, out_dim))


def kernel(x, edge_index, edge_weight, batch, W1, b1, W2, b2, W3, b3):
    e = edge_index.shape[1]
    grain = NC * NS * CHUNK * NEBUF
    epad = ((e + grain - 1) // grain) * grain
    pad = epad - e
    src = jnp.concatenate([edge_index[0], jnp.zeros((pad,), jnp.int32)])
    dst = jnp.concatenate([edge_index[1], jnp.zeros((pad,), jnp.int32)])
    w = jnp.concatenate([edge_weight, jnp.zeros((pad,), jnp.float32)])
    nct = epad // CHUNK
    edata = jnp.stack(
        [src.reshape(nct, CHUNK),
         dst.reshape(nct, CHUNK),
         lax.bitcast_convert_type(w, jnp.int32).reshape(nct, CHUNK)],
        axis=1)
    zeros = jnp.zeros((NPAD, D), jnp.float32)
    hp = jnp.zeros((NPAD, D), jnp.float32).at[:N].set(x)

    agg = _propagate_sc(hp, edata, zeros, epad)
    hp = _dense_tc(agg, hp, W1, b1, True, True)
    agg = _propagate_sc(hp, edata, zeros, epad)
    hp = _dense_tc(agg, hp, W2, b2, True, True)
    agg = _propagate_sc(hp, edata, zeros, epad)
    return _dense_tc(agg, hp, W3, b3, False, False)
